# Initial kernel scaffold; baseline (speedup 1.0000x reference)
import jax
import jax.numpy as jnp
from jax.experimental import pallas as pl


def kernel(x, edge_index, edge_attr, batch, params):
    raise NotImplementedError("write your pallas kernel here")


# speedup vs baseline: 8.3864x; 8.3864x over previous
import functools

import jax
import jax.numpy as jnp
from jax import lax
from jax.experimental import pallas as pl
from jax.experimental.pallas import tpu as pltpu
from jax.experimental.pallas import tpu_sc as plsc

_N = 10000
_E = 640000
_D = 110
_DP = 128
_IN = 3
_SP = 8

_NT = 32
_K = 128
_FULL = _E // (_NT * _K)
_TAIL_BASE = _NT * _FULL * _K
_TAIL_CHUNKS = (_E - _TAIL_BASE) // _K
_PREC = None
_RPS = 640
_ZR = 80


def _sc_gather_scatter_add(m, src, dst):
    mesh = plsc.VectorSubcoreMesh(core_axis_name="c", subcore_axis_name="s")

    @functools.partial(
        pl.kernel,
        out_type=jax.ShapeDtypeStruct((2, _N, _DP), jnp.float32),
        mesh=mesh,
        scratch_types=[
            pltpu.VMEM((_K,), jnp.int32),
            pltpu.VMEM((_K,), jnp.int32),
            pltpu.VMEM((_K, _DP), jnp.float32),
            pltpu.VMEM((_ZR, _DP), jnp.float32),
            pltpu.VMEM_SHARED((_N, _DP), jnp.float32),
            pltpu.SemaphoreType.DMA,
        ],
    )
    def k(m_hbm, src_hbm, dst_hbm, out_hbm, sidx, didx, rows, zbuf, agg_sh, sem):
        cid = lax.axis_index("c")
        sid = lax.axis_index("s")
        wid = cid * 16 + sid

        def zrow(i, carry):
            for g in range(_DP // 16):
                zbuf[i, pl.ds(g * 16, 16)] = jnp.zeros((16,), jnp.float32)
            return carry

        lax.fori_loop(0, _ZR, zrow, 0)
        nz = jnp.where(sid < 15, 8, 5)

        def zcopy(t, carry):
            pltpu.sync_copy(zbuf, agg_sh.at[pl.ds(sid * _RPS + t * _ZR, _ZR)])
            return carry

        lax.fori_loop(0, nz, zcopy, 0)
        plsc.subcore_barrier()

        nchunks = _FULL + jnp.where(wid < _TAIL_CHUNKS, 1, 0)

        def body(j, carry):
            base = jnp.where(j < _FULL,
                             wid * (_FULL * _K) + j * _K,
                             _TAIL_BASE + wid * _K)
            pltpu.sync_copy(src_hbm.at[pl.ds(base, _K)], sidx)
            pltpu.sync_copy(dst_hbm.at[pl.ds(base, _K)], didx)
            pltpu.async_copy(m_hbm.at[sidx], rows, sem).wait()
            pltpu.sync_copy(rows, agg_sh.at[didx], add=True)
            return carry

        lax.fori_loop(0, nchunks, body, 0)
        plsc.subcore_barrier()

        @pl.when(sid < 15)
        def _():
            pltpu.sync_copy(agg_sh.at[pl.ds(sid * _RPS, _RPS)],
                            out_hbm.at[cid, pl.ds(sid * _RPS, _RPS)])

        @pl.when(sid == 15)
        def _():
            pltpu.sync_copy(agg_sh.at[pl.ds(15 * _RPS, _N - 15 * _RPS)],
                            out_hbm.at[cid, pl.ds(15 * _RPS, _N - 15 * _RPS)])

    return k(m, src, dst)


def _embed_body(x_ref, emb_ref, wp_ref, bp_ref, h_ref, m_ref, s_ref):
    xc = x_ref[...]
    h = jnp.zeros((_N, _DP), jnp.float32)
    for v in range(_IN):
        onehot = (xc == v).astype(jnp.float32)
        h = h + onehot * emb_ref[v:v + 1, :]
    h_ref[...] = h
    m_ref[...] = jnp.maximum(h, 0.0)
    s_ref[...] = (jnp.dot(h, wp_ref[...], precision=_PREC,
                          preferred_element_type=jnp.float32)
                  + bp_ref[...])


def _embed_call(xc, emb, wp, bp):
    return pl.pallas_call(
        _embed_body,
        out_shape=(
            jax.ShapeDtypeStruct((_N, _DP), jnp.float32),
            jax.ShapeDtypeStruct((_N, _DP), jnp.float32),
            jax.ShapeDtypeStruct((_N, _SP), jnp.float32),
        ),
    )(xc, emb, wp, bp)


def _bn(y, g, b):
    mu = jnp.mean(y, axis=0, keepdims=True)
    var = jnp.mean((y - mu) * (y - mu), axis=0, keepdims=True)
    return g * (y - mu) / jnp.sqrt(var + 1e-5) + b


def _dense_body(h_ref, aggp_ref, eps_ref, w1_ref, b1_ref, g1_ref, be1_ref,
                w2_ref, b2_ref, g2_ref, be2_ref, wp_ref, bp_ref, s_in_ref,
                h_out_ref, m_out_ref, s_out_ref):
    h = h_ref[...]
    agg = aggp_ref[0] + aggp_ref[1]
    z = (1.0 + eps_ref[0, 0]) * h + agg
    y1 = jnp.dot(z, w1_ref[...], precision=_PREC,
                 preferred_element_type=jnp.float32) + b1_ref[...]
    z1 = jnp.maximum(_bn(y1, g1_ref[...], be1_ref[...]), 0.0)
    y2 = jnp.dot(z1, w2_ref[...], precision=_PREC,
                 preferred_element_type=jnp.float32) + b2_ref[...]
    hnew = jnp.maximum(_bn(y2, g2_ref[...], be2_ref[...]), 0.0)
    h_out = h + hnew
    h_out_ref[...] = h_out
    m_out_ref[...] = jnp.maximum(h_out, 0.0)
    s_out_ref[...] = (s_in_ref[...]
                      + jnp.dot(h_out, wp_ref[...], precision=_PREC,
                                preferred_element_type=jnp.float32)
                      + bp_ref[...])


def _dense_call(h, aggp, eps, lw, wp, bp, score):
    return pl.pallas_call(
        _dense_body,
        out_shape=(
            jax.ShapeDtypeStruct((_N, _DP), jnp.float32),
            jax.ShapeDtypeStruct((_N, _DP), jnp.float32),
            jax.ShapeDtypeStruct((_N, _SP), jnp.float32),
        ),
        in_specs=[
            pl.BlockSpec((_N, _DP), lambda: (0, 0)),
            pl.BlockSpec((2, _N, _DP), lambda: (0, 0, 0)),
            pl.BlockSpec(memory_space=pltpu.SMEM),
            pl.BlockSpec((_DP, _DP), lambda: (0, 0)),
            pl.BlockSpec((1, _DP), lambda: (0, 0)),
            pl.BlockSpec((1, _DP), lambda: (0, 0)),
            pl.BlockSpec((1, _DP), lambda: (0, 0)),
            pl.BlockSpec((_DP, _DP), lambda: (0, 0)),
            pl.BlockSpec((1, _DP), lambda: (0, 0)),
            pl.BlockSpec((1, _DP), lambda: (0, 0)),
            pl.BlockSpec((1, _DP), lambda: (0, 0)),
            pl.BlockSpec((_DP, _SP), lambda: (0, 0)),
            pl.BlockSpec((1, _SP), lambda: (0, 0)),
            pl.BlockSpec((_N, _SP), lambda: (0, 0)),
        ],
    )(h, aggp, eps, lw['W1'], lw['b1'], lw['bn1_g'], lw['bn1_b'],
      lw['W2'], lw['b2'], lw['bn_g'], lw['bn_b'], wp, bp, score)


def _pad_cols(a, width):
    return jnp.pad(a, ((0, 0), (0, width - a.shape[1])))


def _pad_vec(a, width):
    return jnp.pad(a, (0, width - a.shape[0])).reshape(1, width)


def kernel(x, edge_index, edge_attr, batch, params):
    del edge_attr, batch
    src = edge_index[0]
    dst = edge_index[1]
    xc = x.reshape(_N, 1).astype(jnp.int32)

    emb = jnp.pad(params['emb'], ((0, 8 - _IN), (0, _DP - _D)))
    preds = params['preds']
    wps = [_pad_cols(jnp.pad(p['W'], ((0, _DP - _D), (0, 0))), _SP)
           for p in preds]
    bp0 = _pad_vec(preds[0]['b'], _SP)
    for p in preds[1:]:
        bp0 = bp0 + _pad_vec(p['b'], _SP)
    bp_zero = jnp.zeros((1, _SP), jnp.float32)

    h, m, score = _embed_call(xc, emb, wps[0], bp0)

    for i, lp in enumerate(params['layers']):
        lw = {
            'W1': jnp.pad(lp['W1'], ((0, _DP - _D), (0, _DP - _D))),
            'b1': _pad_vec(lp['b1'], _DP),
            'bn1_g': _pad_vec(lp['bn1_g'], _DP),
            'bn1_b': _pad_vec(lp['bn1_b'], _DP),
            'W2': jnp.pad(lp['W2'], ((0, _DP - _D), (0, _DP - _D))),
            'b2': _pad_vec(lp['b2'], _DP),
            'bn_g': _pad_vec(lp['bn_g'], _DP),
            'bn_b': _pad_vec(lp['bn_b'], _DP),
        }
        eps = lp['eps'].reshape(1, 1)
        aggp = _sc_gather_scatter_add(m, src, dst)
        h, m, score = _dense_call(h, aggp, eps, lw, wps[i + 1], bp_zero, score)

    return score[:, :2]


# speedup vs baseline: 11.1680x; 1.3317x over previous
import functools

import numpy as np

import jax
import jax.numpy as jnp
from jax import lax
from jax.experimental import pallas as pl
from jax.experimental.pallas import tpu as pltpu
from jax.experimental.pallas import tpu_sc as plsc

_N = 10000
_E = 640000
_D = 110
_DP = 128
_IN = 3
_SP = 8

_NT = 32
_K = 80
_EPT = _E // _NT
_NCH = _EPT // _K
_PREC = None
_RPS = 640
_ZR = 80


def _sc_gather_scatter_add(m, src, dst):
    mesh = plsc.VectorSubcoreMesh(core_axis_name="c", subcore_axis_name="s")

    @functools.partial(
        pl.kernel,
        out_type=jax.ShapeDtypeStruct((2, _N, _DP), jnp.float32),
        mesh=mesh,
        scratch_types=[
            pltpu.VMEM((_K,), jnp.int32),
            pltpu.VMEM((_K,), jnp.int32),
            pltpu.VMEM((_K,), jnp.int32),
            pltpu.VMEM((_K,), jnp.int32),
            pltpu.VMEM((_K, _DP), jnp.float32),
            pltpu.VMEM((_K, _DP), jnp.float32),
            pltpu.VMEM((_ZR, _DP), jnp.float32),
            pltpu.VMEM_SHARED((_N, _DP), jnp.float32),
            pltpu.SemaphoreType.DMA,
            pltpu.SemaphoreType.DMA,
        ],
    )
    def k(m_hbm, src_hbm, dst_hbm, out_hbm, sidx_a, didx_a, sidx_b, didx_b,
          rows_a, rows_b, zbuf, agg_sh, sem_a, sem_b):
        cid = lax.axis_index("c")
        sid = lax.axis_index("s")
        wid = cid * 16 + sid

        def zrow(i, carry):
            for g in range(_DP // 16):
                zbuf[i, pl.ds(g * 16, 16)] = jnp.zeros((16,), jnp.float32)
            return carry

        lax.fori_loop(0, _ZR, zrow, 0)
        nz = jnp.where(sid < 15, 8, 5)

        def zcopy(t, carry):
            pltpu.sync_copy(zbuf, agg_sh.at[pl.ds(sid * _RPS + t * _ZR, _ZR)])
            return carry

        lax.fori_loop(0, nz, zcopy, 0)
        plsc.subcore_barrier()

        ebase = wid * _EPT

        def fire(c, sidx, didx, rows, sem):
            pltpu.sync_copy(src_hbm.at[pl.ds(ebase + c * _K, _K)], sidx)
            pltpu.sync_copy(dst_hbm.at[pl.ds(ebase + c * _K, _K)], didx)
            pltpu.async_copy(m_hbm.at[sidx], rows, sem)

        def drain(sidx, rows, sem):
            pltpu.make_async_copy(m_hbm.at[sidx], rows, sem).wait()

        fire(0, sidx_a, didx_a, rows_a, sem_a)

        def pair(p, carry):
            c0 = 2 * p
            fire(c0 + 1, sidx_b, didx_b, rows_b, sem_b)
            drain(sidx_a, rows_a, sem_a)
            pltpu.sync_copy(rows_a, agg_sh.at[didx_a], add=True)

            @pl.when(p < _NCH // 2 - 1)
            def _():
                fire(c0 + 2, sidx_a, didx_a, rows_a, sem_a)

            drain(sidx_b, rows_b, sem_b)
            pltpu.sync_copy(rows_b, agg_sh.at[didx_b], add=True)
            return carry

        lax.fori_loop(0, _NCH // 2, pair, 0)
        plsc.subcore_barrier()

        @pl.when(sid < 15)
        def _():
            pltpu.sync_copy(agg_sh.at[pl.ds(sid * _RPS, _RPS)],
                            out_hbm.at[cid, pl.ds(sid * _RPS, _RPS)])

        @pl.when(sid == 15)
        def _():
            pltpu.sync_copy(agg_sh.at[pl.ds(15 * _RPS, _N - 15 * _RPS)],
                            out_hbm.at[cid, pl.ds(15 * _RPS, _N - 15 * _RPS)])

    return k(m, src, dst)


def _embed_body(x_ref, emb_ref, wp_ref, bp_ref, h_ref, m_ref, s_ref):
    xc = x_ref[...]
    h = jnp.zeros((_N, _DP), jnp.float32)
    for v in range(_IN):
        onehot = (xc == v).astype(jnp.float32)
        h = h + onehot * emb_ref[v:v + 1, :]
    h_ref[...] = h
    m_ref[...] = jnp.maximum(h, 0.0)
    s_ref[...] = (jnp.dot(h, wp_ref[...], precision=_PREC,
                          preferred_element_type=jnp.float32)
                  + bp_ref[...])


def _embed_call(xc, emb, wp, bp):
    return pl.pallas_call(
        _embed_body,
        out_shape=(
            jax.ShapeDtypeStruct((_N, _DP), jnp.float32),
            jax.ShapeDtypeStruct((_N, _DP), jnp.float32),
            jax.ShapeDtypeStruct((_N, _SP), jnp.float32),
        ),
    )(xc, emb, wp, bp)


_INV_N = np.float32(1.0 / _N)


def _colmean(y, mu=None):
    def slab(a):
        if mu is None:
            return a
        d = a - mu
        return d * d

    acc = slab(y[0:128])
    for g in range(1, 78):
        acc = acc + slab(y[g * 128:(g + 1) * 128])
    tail = jnp.concatenate(
        [slab(y[9984:10000]), jnp.zeros((112, y.shape[1]), jnp.float32)],
        axis=0)
    acc = acc + tail
    total = acc[0:8]
    for j in range(1, 16):
        total = total + acc[8 * j:8 * (j + 1)]
    s = total[0:4] + total[4:8]
    s = s[0:2] + s[2:4]
    s = s[0:1] + s[1:2]
    return s * _INV_N


def _bn(y, g, b):
    mu = _colmean(y)
    var = _colmean(y, mu)
    return g * (y - mu) / jnp.sqrt(var + 1e-5) + b


def _dense_body(h_ref, aggp_ref, eps_ref, w1_ref, b1_ref, g1_ref, be1_ref,
                w2_ref, b2_ref, g2_ref, be2_ref, wp_ref, bp_ref, s_in_ref,
                h_out_ref, m_out_ref, s_out_ref):
    h = h_ref[...]
    agg = aggp_ref[0] + aggp_ref[1]
    z = (1.0 + eps_ref[0, 0]) * h + agg
    y1 = jnp.dot(z, w1_ref[...], precision=_PREC,
                 preferred_element_type=jnp.float32) + b1_ref[...]
    z1 = jnp.maximum(_bn(y1, g1_ref[...], be1_ref[...]), 0.0)
    y2 = jnp.dot(z1, w2_ref[...], precision=_PREC,
                 preferred_element_type=jnp.float32) + b2_ref[...]
    hnew = jnp.maximum(_bn(y2, g2_ref[...], be2_ref[...]), 0.0)
    h_out = h + hnew
    h_out_ref[...] = h_out
    m_out_ref[...] = jnp.maximum(h_out, 0.0)
    s_out_ref[...] = (s_in_ref[...]
                      + jnp.dot(h_out, wp_ref[...], precision=_PREC,
                                preferred_element_type=jnp.float32)
                      + bp_ref[...])


def _dense_call(h, aggp, eps, lw, wp, bp, score):
    return pl.pallas_call(
        _dense_body,
        out_shape=(
            jax.ShapeDtypeStruct((_N, _DP), jnp.float32),
            jax.ShapeDtypeStruct((_N, _DP), jnp.float32),
            jax.ShapeDtypeStruct((_N, _SP), jnp.float32),
        ),
        in_specs=[
            pl.BlockSpec((_N, _DP), lambda: (0, 0)),
            pl.BlockSpec((2, _N, _DP), lambda: (0, 0, 0)),
            pl.BlockSpec(memory_space=pltpu.SMEM),
            pl.BlockSpec((_DP, _DP), lambda: (0, 0)),
            pl.BlockSpec((1, _DP), lambda: (0, 0)),
            pl.BlockSpec((1, _DP), lambda: (0, 0)),
            pl.BlockSpec((1, _DP), lambda: (0, 0)),
            pl.BlockSpec((_DP, _DP), lambda: (0, 0)),
            pl.BlockSpec((1, _DP), lambda: (0, 0)),
            pl.BlockSpec((1, _DP), lambda: (0, 0)),
            pl.BlockSpec((1, _DP), lambda: (0, 0)),
            pl.BlockSpec((_DP, _SP), lambda: (0, 0)),
            pl.BlockSpec((1, _SP), lambda: (0, 0)),
            pl.BlockSpec((_N, _SP), lambda: (0, 0)),
        ],
    )(h, aggp, eps, lw['W1'], lw['b1'], lw['bn1_g'], lw['bn1_b'],
      lw['W2'], lw['b2'], lw['bn_g'], lw['bn_b'], wp, bp, score)


def _pad_cols(a, width):
    return jnp.pad(a, ((0, 0), (0, width - a.shape[1])))


def _pad_vec(a, width):
    return jnp.pad(a, (0, width - a.shape[0])).reshape(1, width)


def kernel(x, edge_index, edge_attr, batch, params):
    del edge_attr, batch
    src = edge_index[0]
    dst = edge_index[1]
    xc = x.reshape(_N, 1).astype(jnp.int32)

    emb = jnp.pad(params['emb'], ((0, 8 - _IN), (0, _DP - _D)))
    preds = params['preds']
    wps = [_pad_cols(jnp.pad(p['W'], ((0, _DP - _D), (0, 0))), _SP)
           for p in preds]
    bp0 = _pad_vec(preds[0]['b'], _SP)
    for p in preds[1:]:
        bp0 = bp0 + _pad_vec(p['b'], _SP)
    bp_zero = jnp.zeros((1, _SP), jnp.float32)

    h, m, score = _embed_call(xc, emb, wps[0], bp0)

    for i, lp in enumerate(params['layers']):
        lw = {
            'W1': jnp.pad(lp['W1'], ((0, _DP - _D), (0, _DP - _D))),
            'b1': _pad_vec(lp['b1'], _DP),
            'bn1_g': _pad_vec(lp['bn1_g'], _DP),
            'bn1_b': _pad_vec(lp['bn1_b'], _DP),
            'W2': jnp.pad(lp['W2'], ((0, _DP - _D), (0, _DP - _D))),
            'b2': _pad_vec(lp['b2'], _DP),
            'bn_g': _pad_vec(lp['bn_g'], _DP),
            'bn_b': _pad_vec(lp['bn_b'], _DP),
        }
        eps = lp['eps'].reshape(1, 1)
        aggp = _sc_gather_scatter_add(m, src, dst)
        h, m, score = _dense_call(h, aggp, eps, lw, wps[i + 1], bp_zero, score)

    return score[:, :2]
